# Initial kernel scaffold; baseline (speedup 1.0000x reference)
import jax
import jax.numpy as jnp
from jax.experimental import pallas as pl


def kernel(x, edge_index, edge_type, src_idx, dst_idx, W_emb, b_emb, edge_table, W_edge0, b_edge0, Wl0, bl0, Wr0, W_edge1, b_edge1, Wl1, bl1, Wr1, W1, b1, W2, b2):
    raise NotImplementedError("write your pallas kernel here")


# speedup vs baseline: 6.2165x; 6.2165x over previous
import functools

import jax
import jax.numpy as jnp
from jax import lax
from jax.experimental import pallas as pl
from jax.experimental.pallas import tpu as pltpu
from jax.experimental.pallas import tpu_sc as plsc

_N = 10000
_E = 320000
_H = 128
_B = 8192

_NC = 2
_NS = 16
_NW = _NC * _NS

_NP = 10240
_RPT = _NP // _NS
_EPT = _E // _NW
_CH = 80
_NCHUNK = _EPT // _CH
_CW = 16
_SR = 128

_ROWS_TC = 1000
_ROWS_MLP = 1024


def _seg_body(h_hbm, src_hbm, dst_hbm, z_hbm, agg_out,
              src_v, dst_v, rows_v, stage_v, sem, agg_sh):
    c = lax.axis_index("c")
    s = lax.axis_index("s")
    wid = s * _NC + c
    r0 = s * _RPT
    pltpu.sync_copy(z_hbm, stage_v)

    def zero_slab(j, carry):
        pltpu.sync_copy(stage_v, agg_sh.at[pl.ds(r0 + j * _SR, _SR)])
        return carry
    lax.fori_loop(0, _RPT // _SR, zero_slab, 0)
    plsc.subcore_barrier()

    ebase = wid * _EPT

    def body(i, carry):
        base = ebase + i * _CH
        pltpu.sync_copy(src_hbm.at[pl.ds(base, _CH)], src_v)
        pltpu.sync_copy(dst_hbm.at[pl.ds(base, _CH)], dst_v)
        pltpu.async_copy(h_hbm.at[src_v], rows_v, sem).wait()
        pltpu.sync_copy(rows_v, agg_sh.at[dst_v], add=True)
        return carry
    lax.fori_loop(0, _NCHUNK, body, 0)
    plsc.subcore_barrier()

    def copy_out(j, carry):
        pltpu.sync_copy(agg_sh.at[pl.ds(r0 + j * _SR, _SR)], stage_v)
        pltpu.sync_copy(stage_v,
                        agg_out.at[pl.ds(c * _NP + r0 + j * _SR, _SR)])
        return carry
    lax.fori_loop(0, _RPT // _SR, copy_out, 0)


_seg = pl.kernel(
    _seg_body,
    out_type=jax.ShapeDtypeStruct((_NC * _NP, _H), jnp.float32),
    mesh=plsc.VectorSubcoreMesh(core_axis_name="c", subcore_axis_name="s"),
    scratch_types=(pltpu.VMEM((_CH,), jnp.int32),
                   pltpu.VMEM((_CH,), jnp.int32),
                   pltpu.VMEM((_CH, _H), jnp.float32),
                   pltpu.VMEM((_SR, _H), jnp.float32),
                   pltpu.SemaphoreType.DMA,
                   pltpu.VMEM_SHARED((_NP, _H), jnp.float32)))


def _cnt_body(dst_hbm, cnt_out, dst_v, ones_v, stage_c, cnt_sh):
    c = lax.axis_index("c")
    s = lax.axis_index("s")
    wid = s * _NC + c
    r0 = s * _RPT

    def fill(i, carry):
        stage_c[i] = jnp.zeros((_CW,), jnp.float32)
        ones_v[i] = jnp.ones((_CW,), jnp.float32)
        return carry
    lax.fori_loop(0, _SR, fill, 0)

    def zero_slab(j, carry):
        pltpu.sync_copy(stage_c, cnt_sh.at[pl.ds(r0 + j * _SR, _SR)])
        return carry
    lax.fori_loop(0, _RPT // _SR, zero_slab, 0)
    plsc.subcore_barrier()

    ebase = wid * _EPT

    def body(i, carry):
        base = ebase + i * _CH
        pltpu.sync_copy(dst_hbm.at[pl.ds(base, _CH)], dst_v)
        pltpu.sync_copy(ones_v.at[pl.ds(0, _CH)], cnt_sh.at[dst_v], add=True)
        return carry
    lax.fori_loop(0, _NCHUNK, body, 0)
    plsc.subcore_barrier()

    def copy_out(j, carry):
        pltpu.sync_copy(cnt_sh.at[pl.ds(r0 + j * _SR, _SR)], stage_c)
        pltpu.sync_copy(stage_c,
                        cnt_out.at[pl.ds(c * _NP + r0 + j * _SR, _SR)])
        return carry
    lax.fori_loop(0, _RPT // _SR, copy_out, 0)


_cnt_call = pl.kernel(
    _cnt_body,
    out_type=jax.ShapeDtypeStruct((_NC * _NP, _CW), jnp.float32),
    mesh=plsc.VectorSubcoreMesh(core_axis_name="c", subcore_axis_name="s"),
    scratch_types=(pltpu.VMEM((_CH,), jnp.int32),
                   pltpu.VMEM((max(_CH, _SR), _CW), jnp.float32),
                   pltpu.VMEM((_SR, _CW), jnp.float32),
                   pltpu.VMEM_SHARED((_NP, _CW), jnp.float32)))


def _pair_gather_body(h_hbm, idx_hbm, out_hbm, idx_v, rows_v, sem):
    c = lax.axis_index("c")
    s = lax.axis_index("s")
    wid = s * _NC + c
    gpt = (2 * _B) // _NW
    gch = 128
    def body(i, carry):
        base = wid * gpt + i * gch
        pltpu.sync_copy(idx_hbm.at[pl.ds(base, gch)], idx_v)
        pltpu.async_copy(h_hbm.at[idx_v], rows_v, sem).wait()
        pltpu.sync_copy(rows_v, out_hbm.at[pl.ds(base, gch)])
        return carry
    lax.fori_loop(0, gpt // gch, body, 0)


_pair_gather = pl.kernel(
    _pair_gather_body,
    out_type=jax.ShapeDtypeStruct((2 * _B, _H), jnp.float32),
    mesh=plsc.VectorSubcoreMesh(core_axis_name="c", subcore_axis_name="s"),
    scratch_types=(pltpu.VMEM((128,), jnp.int32),
                   pltpu.VMEM((128, _H), jnp.float32),
                   pltpu.SemaphoreType.DMA))


_DN = (((1,), (1,)), ((), ()))


def _emb_body(x_ref, w_ref, b_ref, o_ref):
    o_ref[...] = lax.dot_general(x_ref[...], w_ref[...], _DN,
                                 preferred_element_type=jnp.float32) + b_ref[...]


def _emb_call(x, w, b):
    grid = _N // _ROWS_TC
    return pl.pallas_call(
        _emb_body,
        grid=(grid,),
        in_specs=[pl.BlockSpec((_ROWS_TC, _H), lambda i: (i, 0)),
                  pl.BlockSpec((_H, _H), lambda i: (0, 0)),
                  pl.BlockSpec((1, _H), lambda i: (0, 0))],
        out_specs=pl.BlockSpec((_ROWS_TC, _H), lambda i: (i, 0)),
        out_shape=jax.ShapeDtypeStruct((_N, _H), jnp.float32),
    )(x, w, b)


def _sage_body(a0_ref, a1_ref, c0_ref, c1_ref, h_ref, wl_ref, bl_ref, wr_ref,
               o_ref):
    h = h_ref[...]
    agg = a0_ref[...] + a1_ref[...] + h
    cnt = c0_ref[...][:, :1] + c1_ref[...][:, :1] + 1.0
    mean = agg / cnt
    o = (lax.dot_general(mean, wl_ref[...], _DN,
                         preferred_element_type=jnp.float32)
         + bl_ref[...]
         + lax.dot_general(h, wr_ref[...], _DN,
                           preferred_element_type=jnp.float32))
    o_ref[...] = jnp.maximum(o, 0.0)


def _sage_call(a0, a1, c0, c1, h, wl, bl, wr):
    grid = _N // _ROWS_TC
    rspec = pl.BlockSpec((_ROWS_TC, _H), lambda i: (i, 0))
    cspec = pl.BlockSpec((_ROWS_TC, _CW), lambda i: (i, 0))
    wspec = pl.BlockSpec((_H, _H), lambda i: (0, 0))
    return pl.pallas_call(
        _sage_body,
        grid=(grid,),
        in_specs=[rspec, rspec, cspec, cspec, rspec, wspec,
                  pl.BlockSpec((1, _H), lambda i: (0, 0)), wspec],
        out_specs=rspec,
        out_shape=jax.ShapeDtypeStruct((_N, _H), jnp.float32),
    )(a0, a1, c0, c1, h, wl, bl, wr)


def _mlp_body(s_ref, d_ref, w1_ref, b1_ref, w2_ref, b2_ref, o_ref):
    s = s_ref[...]
    d = d_ref[...]
    w1 = w1_ref[...]
    hid = (lax.dot_general(s, w1[:, 0:_H], _DN,
                           preferred_element_type=jnp.float32)
           + lax.dot_general(d, w1[:, _H:2 * _H], _DN,
                             preferred_element_type=jnp.float32)
           + lax.dot_general(s * d, w1[:, 2 * _H:3 * _H], _DN,
                             preferred_element_type=jnp.float32)
           + lax.dot_general(jnp.abs(s - d), w1[:, 3 * _H:4 * _H], _DN,
                             preferred_element_type=jnp.float32)
           + b1_ref[...])
    hid = jnp.maximum(hid, 0.0)
    z = jnp.sum(hid * w2_ref[...], axis=1, keepdims=True) + b2_ref[...]
    o_ref[...] = 1.0 / (1.0 + jnp.exp(-z))


def _mlp_call(s, d, w1, b1, w2, b2):
    grid = _B // _ROWS_MLP
    rspec = pl.BlockSpec((_ROWS_MLP, _H), lambda i: (i, 0))
    return pl.pallas_call(
        _mlp_body,
        grid=(grid,),
        in_specs=[rspec, rspec,
                  pl.BlockSpec((_H, 4 * _H), lambda i: (0, 0)),
                  pl.BlockSpec((1, _H), lambda i: (0, 0)),
                  pl.BlockSpec((1, _H), lambda i: (0, 0)),
                  pl.BlockSpec((1, 1), lambda i: (0, 0))],
        out_specs=pl.BlockSpec((_ROWS_MLP, 1), lambda i: (i, 0)),
        out_shape=jax.ShapeDtypeStruct((_B, 1), jnp.float32),
    )(s, d, w1, b1, w2, b2)


def kernel(x, edge_index, edge_type, src_idx, dst_idx, W_emb, b_emb,
           edge_table, W_edge0, b_edge0, Wl0, bl0, Wr0,
           W_edge1, b_edge1, Wl1, bl1, Wr1, W1, b1, W2, b2):
    src = edge_index[0]
    dst = edge_index[1]
    z_h = jnp.zeros((_SR, _H), jnp.float32)

    cnt_p = _cnt_call(dst)
    h0 = _emb_call(x, W_emb, b_emb.reshape(1, _H))

    agg_p = _seg(h0, src, dst, z_h)
    c0 = cnt_p[:_N]
    c1 = cnt_p[_NP:_NP + _N]
    h1 = _sage_call(agg_p[:_N], agg_p[_NP:_NP + _N], c0, c1, h0,
                    Wl0, bl0.reshape(1, _H), Wr0)

    agg_p2 = _seg(h1, src, dst, z_h)
    h2 = _sage_call(agg_p2[:_N], agg_p2[_NP:_NP + _N], c0, c1, h1,
                    Wl1, bl1.reshape(1, _H), Wr1)

    idx_all = jnp.concatenate([src_idx, dst_idx], axis=0)
    pair_emb = _pair_gather(h2, idx_all)
    out = _mlp_call(pair_emb[:_B], pair_emb[_B:], W1, b1.reshape(1, _H),
                    W2, b2.reshape(1, 1))
    return jnp.squeeze(out, axis=-1)


# speedup vs baseline: 11.5368x; 1.8558x over previous
import jax
import jax.numpy as jnp
from jax import lax
from jax.experimental import pallas as pl
from jax.experimental.pallas import tpu as pltpu
from jax.experimental.pallas import tpu_sc as plsc

_N = 10000
_E = 320000
_H = 128
_B = 8192

_NC = 2
_NS = 16
_NW = _NC * _NS

_NP = 10240
_RPT = _NP // _NS
_EPT = _E // _NW
_CH = 80
_CPT = _EPT // _CH
_CW = 16
_CCH = 125
_CCPT = _EPT // _CCH
_SR = 128

_ROWS_TC = 1000
_ROWS_MLP = 1024


def _seg_body(h_hbm, se_hbm, de_hbm, z_hbm, agg_out,
              src_a, src_b, dst_a, dst_b, rows_a, rows_b,
              sem_sa, sem_sb, sem_da, sem_db, sem_ga, sem_gb, agg_sh):
    c = lax.axis_index("c")
    s = lax.axis_index("s")
    wid = s * _NC + c
    r0 = s * _RPT
    ebase = wid * _EPT
    last = _CPT - 1

    def sidx(j, buf, sem):
        j = jnp.minimum(j, last)
        pltpu.make_async_copy(se_hbm.at[pl.ds(ebase + j * _CH, _CH)],
                              buf, sem).start()

    def didx(j, buf, sem):
        j = jnp.minimum(j, last)
        pltpu.make_async_copy(de_hbm.at[pl.ds(ebase + j * _CH, _CH)],
                              buf, sem).start()

    def wait_si(j, buf, sem):
        j = jnp.minimum(j, last)
        pltpu.make_async_copy(se_hbm.at[pl.ds(ebase + j * _CH, _CH)],
                              buf, sem).wait()

    def wait_di(j, buf, sem):
        j = jnp.minimum(j, last)
        pltpu.make_async_copy(de_hbm.at[pl.ds(ebase + j * _CH, _CH)],
                              buf, sem).wait()

    def start_g(rows, sbuf, sem):
        pltpu.make_async_copy(h_hbm.at[sbuf], rows, sem).start()

    def wait_g(rows, sbuf, sem):
        pltpu.make_async_copy(h_hbm.at[sbuf], rows, sem).wait()

    def scat(rows, dbuf):
        pltpu.sync_copy(rows, agg_sh.at[dbuf], add=True)

    pltpu.sync_copy(z_hbm, rows_a)

    def zero_slab(j, carry):
        pltpu.sync_copy(rows_a, agg_sh.at[pl.ds(r0 + j * _CH, _CH)])
        return carry
    lax.fori_loop(0, _RPT // _CH, zero_slab, 0)
    plsc.subcore_barrier()

    sidx(0, src_a, sem_sa)
    didx(0, dst_a, sem_da)
    wait_si(0, src_a, sem_sa)
    start_g(rows_a, src_a, sem_ga)
    sidx(1, src_b, sem_sb)
    didx(1, dst_b, sem_db)

    def body(k, carry):
        wait_si(2 * k + 1, src_b, sem_sb)
        start_g(rows_b, src_b, sem_gb)
        wait_g(rows_a, src_a, sem_ga)
        sidx(2 * k + 2, src_a, sem_sa)
        wait_di(2 * k, dst_a, sem_da)
        scat(rows_a, dst_a)
        didx(2 * k + 2, dst_a, sem_da)
        wait_si(2 * k + 2, src_a, sem_sa)
        start_g(rows_a, src_a, sem_ga)
        wait_g(rows_b, src_b, sem_gb)
        sidx(2 * k + 3, src_b, sem_sb)
        wait_di(2 * k + 1, dst_b, sem_db)
        scat(rows_b, dst_b)
        didx(2 * k + 3, dst_b, sem_db)
        return carry
    lax.fori_loop(0, (_CPT - 1) // 2, body, 0)
    wait_g(rows_a, src_a, sem_ga)
    wait_di(last, dst_a, sem_da)
    scat(rows_a, dst_a)
    wait_si(last, src_b, sem_sb)
    wait_di(last, dst_b, sem_db)
    plsc.subcore_barrier()

    def copy_out(j, carry):
        pltpu.sync_copy(agg_sh.at[pl.ds(r0 + j * _CH, _CH)], rows_a)
        pltpu.sync_copy(rows_a,
                        agg_out.at[pl.ds(c * _NP + r0 + j * _CH, _CH)])
        return carry
    lax.fori_loop(0, _RPT // _CH, copy_out, 0)


_seg = pl.kernel(
    _seg_body,
    out_type=jax.ShapeDtypeStruct((_NC * _NP, _H), jnp.float32),
    mesh=plsc.VectorSubcoreMesh(core_axis_name="c", subcore_axis_name="s"),
    scratch_types=(pltpu.VMEM((_CH,), jnp.int32),
                   pltpu.VMEM((_CH,), jnp.int32),
                   pltpu.VMEM((_CH,), jnp.int32),
                   pltpu.VMEM((_CH,), jnp.int32),
                   pltpu.VMEM((_CH, _H), jnp.float32),
                   pltpu.VMEM((_CH, _H), jnp.float32),
                   pltpu.SemaphoreType.DMA,
                   pltpu.SemaphoreType.DMA,
                   pltpu.SemaphoreType.DMA,
                   pltpu.SemaphoreType.DMA,
                   pltpu.SemaphoreType.DMA,
                   pltpu.SemaphoreType.DMA,
                   pltpu.VMEM_SHARED((_NP, _H), jnp.float32)))


def _cnt_body(dst_hbm, ones_hbm, z_hbm, cnt_out,
              dst_a, dst_b, ones_v, sem_da, sem_db, sem_ka, sem_kb, tab_sh):
    c = lax.axis_index("c")
    s = lax.axis_index("s")
    wid = s * _NC + c
    r0 = s * _RPT
    ebase = wid * _EPT
    last = _CPT - 1

    def didx(j, buf, sem):
        j = jnp.minimum(j, last)
        pltpu.make_async_copy(dst_hbm.at[pl.ds(ebase + j * _CH, _CH)],
                              buf, sem).start()

    def wait_di(j, buf, sem):
        j = jnp.minimum(j, last)
        pltpu.make_async_copy(dst_hbm.at[pl.ds(ebase + j * _CH, _CH)],
                              buf, sem).wait()

    def start_k(dbuf, sem):
        pltpu.make_async_copy(ones_v, tab_sh.at[dbuf], sem).start(add=True)

    def wait_k(dbuf, sem):
        pltpu.make_async_copy(ones_v, tab_sh.at[dbuf], sem).wait()

    pltpu.sync_copy(z_hbm, ones_v)

    def zero_slab(j, carry):
        pltpu.sync_copy(ones_v, tab_sh.at[pl.ds(r0 + j * _CH, _CH)])
        return carry
    lax.fori_loop(0, _RPT // _CH, zero_slab, 0)
    pltpu.sync_copy(ones_hbm, ones_v)
    plsc.subcore_barrier()

    didx(0, dst_a, sem_da)
    didx(1, dst_b, sem_db)
    wait_di(0, dst_a, sem_da)
    start_k(dst_a, sem_ka)
    wait_di(1, dst_b, sem_db)
    start_k(dst_b, sem_kb)

    def body(k, carry):
        wait_k(dst_a, sem_ka)
        didx(2 * k + 2, dst_a, sem_da)
        wait_di(2 * k + 2, dst_a, sem_da)
        start_k(dst_a, sem_ka)
        wait_k(dst_b, sem_kb)
        didx(2 * k + 3, dst_b, sem_db)
        wait_di(2 * k + 3, dst_b, sem_db)
        start_k(dst_b, sem_kb)
        return carry
    lax.fori_loop(0, (_CPT - 1) // 2 - 1, body, 0)
    wait_k(dst_a, sem_ka)
    didx(last, dst_a, sem_da)
    wait_di(last, dst_a, sem_da)
    start_k(dst_a, sem_ka)
    wait_k(dst_b, sem_kb)
    wait_k(dst_a, sem_ka)
    plsc.subcore_barrier()

    def copy_out(j, carry):
        pltpu.sync_copy(tab_sh.at[pl.ds(r0 + j * _CH, _CH)], ones_v)
        pltpu.sync_copy(ones_v,
                        cnt_out.at[pl.ds(c * _NP + r0 + j * _CH, _CH)])
        return carry
    lax.fori_loop(0, _RPT // _CH, copy_out, 0)


_cnt_call = pl.kernel(
    _cnt_body,
    out_type=jax.ShapeDtypeStruct((_NC * _NP, _H), jnp.float32),
    mesh=plsc.VectorSubcoreMesh(core_axis_name="c", subcore_axis_name="s"),
    scratch_types=(pltpu.VMEM((_CH,), jnp.int32),
                   pltpu.VMEM((_CH,), jnp.int32),
                   pltpu.VMEM((_CH, _H), jnp.float32),
                   pltpu.SemaphoreType.DMA,
                   pltpu.SemaphoreType.DMA,
                   pltpu.SemaphoreType.DMA,
                   pltpu.SemaphoreType.DMA,
                   pltpu.VMEM_SHARED((_NP, _H), jnp.float32)))


def _pair_gather_body(h_hbm, idx_hbm, out_hbm, idx_v, rows_v, sem):
    c = lax.axis_index("c")
    s = lax.axis_index("s")
    wid = s * _NC + c
    gpt = (2 * _B) // _NW
    gch = 128
    def body(i, carry):
        base = wid * gpt + i * gch
        pltpu.sync_copy(idx_hbm.at[pl.ds(base, gch)], idx_v)
        pltpu.async_copy(h_hbm.at[idx_v], rows_v, sem).wait()
        pltpu.sync_copy(rows_v, out_hbm.at[pl.ds(base, gch)])
        return carry
    lax.fori_loop(0, gpt // gch, body, 0)


_pair_gather = pl.kernel(
    _pair_gather_body,
    out_type=jax.ShapeDtypeStruct((2 * _B, _H), jnp.float32),
    mesh=plsc.VectorSubcoreMesh(core_axis_name="c", subcore_axis_name="s"),
    scratch_types=(pltpu.VMEM((128,), jnp.int32),
                   pltpu.VMEM((128, _H), jnp.float32),
                   pltpu.SemaphoreType.DMA))


_DN = (((1,), (1,)), ((), ()))


def _emb_body(x_ref, w_ref, b_ref, o_ref):
    o_ref[...] = lax.dot_general(x_ref[...], w_ref[...], _DN,
                                 preferred_element_type=jnp.float32) + b_ref[...]


def _emb_call(x, w, b):
    grid = _N // _ROWS_TC
    return pl.pallas_call(
        _emb_body,
        grid=(grid,),
        in_specs=[pl.BlockSpec((_ROWS_TC, _H), lambda i: (i, 0)),
                  pl.BlockSpec((_H, _H), lambda i: (0, 0)),
                  pl.BlockSpec((1, _H), lambda i: (0, 0))],
        out_specs=pl.BlockSpec((_ROWS_TC, _H), lambda i: (i, 0)),
        out_shape=jax.ShapeDtypeStruct((_N, _H), jnp.float32),
    )(x, w, b)


def _sage_body(a0_ref, a1_ref, c0_ref, c1_ref, h_ref, wl_ref, bl_ref, wr_ref,
               o_ref):
    h = h_ref[...]
    agg = a0_ref[...] + a1_ref[...] + h
    cnt = c0_ref[...][:, :1] + c1_ref[...][:, :1] + 1.0
    mean = agg / cnt
    o = (lax.dot_general(mean, wl_ref[...], _DN,
                         preferred_element_type=jnp.float32)
         + bl_ref[...]
         + lax.dot_general(h, wr_ref[...], _DN,
                           preferred_element_type=jnp.float32))
    o_ref[...] = jnp.maximum(o, 0.0)


def _sage_call(a0, a1, c0, c1, h, wl, bl, wr):
    grid = _N // _ROWS_TC
    rspec = pl.BlockSpec((_ROWS_TC, _H), lambda i: (i, 0))
    cspec = pl.BlockSpec((_ROWS_TC, _H), lambda i: (i, 0))
    wspec = pl.BlockSpec((_H, _H), lambda i: (0, 0))
    return pl.pallas_call(
        _sage_body,
        grid=(grid,),
        in_specs=[rspec, rspec, cspec, cspec, rspec, wspec,
                  pl.BlockSpec((1, _H), lambda i: (0, 0)), wspec],
        out_specs=rspec,
        out_shape=jax.ShapeDtypeStruct((_N, _H), jnp.float32),
    )(a0, a1, c0, c1, h, wl, bl, wr)


def _mlp_body(s_ref, d_ref, w1_ref, b1_ref, w2_ref, b2_ref, o_ref):
    s = s_ref[...]
    d = d_ref[...]
    w1 = w1_ref[...]
    hid = (lax.dot_general(s, w1[:, 0:_H], _DN,
                           preferred_element_type=jnp.float32)
           + lax.dot_general(d, w1[:, _H:2 * _H], _DN,
                             preferred_element_type=jnp.float32)
           + lax.dot_general(s * d, w1[:, 2 * _H:3 * _H], _DN,
                             preferred_element_type=jnp.float32)
           + lax.dot_general(jnp.abs(s - d), w1[:, 3 * _H:4 * _H], _DN,
                             preferred_element_type=jnp.float32)
           + b1_ref[...])
    hid = jnp.maximum(hid, 0.0)
    z = jnp.sum(hid * w2_ref[...], axis=1, keepdims=True) + b2_ref[...]
    o_ref[...] = 1.0 / (1.0 + jnp.exp(-z))


def _mlp_call(s, d, w1, b1, w2, b2):
    grid = _B // _ROWS_MLP
    rspec = pl.BlockSpec((_ROWS_MLP, _H), lambda i: (i, 0))
    return pl.pallas_call(
        _mlp_body,
        grid=(grid,),
        in_specs=[rspec, rspec,
                  pl.BlockSpec((_H, 4 * _H), lambda i: (0, 0)),
                  pl.BlockSpec((1, _H), lambda i: (0, 0)),
                  pl.BlockSpec((1, _H), lambda i: (0, 0)),
                  pl.BlockSpec((1, 1), lambda i: (0, 0))],
        out_specs=pl.BlockSpec((_ROWS_MLP, 1), lambda i: (i, 0)),
        out_shape=jax.ShapeDtypeStruct((_B, 1), jnp.float32),
    )(s, d, w1, b1, w2, b2)


def kernel(x, edge_index, edge_type, src_idx, dst_idx, W_emb, b_emb,
           edge_table, W_edge0, b_edge0, Wl0, bl0, Wr0,
           W_edge1, b_edge1, Wl1, bl1, Wr1, W1, b1, W2, b2):
    src1 = edge_index[0]
    dst1 = edge_index[1]
    z_h = jnp.zeros((_CH, _H), jnp.float32)

    ones_blk = jnp.ones((_CH, _H), jnp.float32)
    cnt_p = _cnt_call(dst1, ones_blk, z_h)
    h0 = _emb_call(x, W_emb, b_emb.reshape(1, _H))

    agg_p = _seg(h0, src1, dst1, z_h)
    c0 = cnt_p[:_N]
    c1 = cnt_p[_NP:_NP + _N]
    h1 = _sage_call(agg_p[:_N], agg_p[_NP:_NP + _N], c0, c1, h0,
                    Wl0, bl0.reshape(1, _H), Wr0)

    agg_p2 = _seg(h1, src1, dst1, z_h)
    h2 = _sage_call(agg_p2[:_N], agg_p2[_NP:_NP + _N], c0, c1, h1,
                    Wl1, bl1.reshape(1, _H), Wr1)

    idx_all = jnp.concatenate([src_idx, dst_idx], axis=0)
    pair_emb = _pair_gather(h2, idx_all)
    out = _mlp_call(pair_emb[:_B], pair_emb[_B:], W1, b1.reshape(1, _H),
                    W2, b2.reshape(1, 1))
    return jnp.squeeze(out, axis=-1)
